# Initial kernel scaffold; baseline (speedup 1.0000x reference)
import jax
import jax.numpy as jnp
from jax.experimental import pallas as pl


def kernel(x, edge_index, W1, b1, gamma, beta, W2, b2):
    raise NotImplementedError("write your pallas kernel here")


# speedup vs baseline: 6.8020x; 6.8020x over previous
import functools

import jax
import jax.numpy as jnp
from jax import lax
from jax.experimental import pallas as pl
from jax.experimental.pallas import tpu as pltpu
from jax.experimental.pallas import tpu_sc as plsc

N = 10000
E = 320000
D = 128
H = 16
C = 64

NUM_CORES = 2
NUM_TILES = 16
NPAD = 10240
HALF = NPAD // NUM_CORES
ROWS_PER_TILE = HALF // NUM_TILES
ZERO_ROW = N
ACC_ROWS = HALF + 16
DUMMY_DST = HALF

EDGES_PER_TILE = E // NUM_TILES
CHUNK = 128
NCHUNK = -(-EDGES_PER_TILE // CHUNK)
EPT_PAD = NCHUNK * CHUNK


def _prep_body(x_ref, w1_ref, src_ref, dst_ref, y_ref, srcg_ref, d0_ref, d1_ref):
    x = x_ref[...]
    w1 = w1_ref[...]
    y_ref[...] = lax.dot_general(x, w1, (((1,), (1,)), ((), ())),
                                 preferred_element_type=jnp.float32,
                                 precision=lax.Precision.HIGHEST)
    src = src_ref[...]
    dst = dst_ref[...]
    srcg_ref[...] = jnp.where(src == dst, ZERO_ROW, src)
    d0_ref[...] = jnp.where(dst < HALF, dst, DUMMY_DST)
    d1_ref[...] = jnp.where(dst >= HALF, dst - HALF, DUMMY_DST)


_prep_call = pl.pallas_call(
    _prep_body,
    out_shape=(
        jax.ShapeDtypeStruct((NPAD, H), jnp.float32),
        jax.ShapeDtypeStruct((E // 128, 128), jnp.int32),
        jax.ShapeDtypeStruct((E // 128, 128), jnp.int32),
        jax.ShapeDtypeStruct((E // 128, 128), jnp.int32),
    ),
)


def _sc_step_body(h_hbm, srcg_hbm, dstl_hbm, out_hbm, acc, sidx, didx, rows, sem):
    c = lax.axis_index("c")
    s = lax.axis_index("s")
    base = c * HALF + s * ROWS_PER_TILE
    pltpu.sync_copy(h_hbm.at[pl.ds(base, ROWS_PER_TILE)],
                    acc.at[pl.ds(s * ROWS_PER_TILE, ROWS_PER_TILE)])
    pltpu.sync_copy(srcg_hbm.at[s], sidx)
    pltpu.sync_copy(dstl_hbm.at[c, s], didx)
    plsc.subcore_barrier()

    def body(j, carry):
        pltpu.async_copy(h_hbm.at[sidx.at[j]], rows, sem).wait()
        pltpu.sync_copy(rows, acc.at[didx.at[j]], add=True)
        return carry

    lax.fori_loop(0, NCHUNK, body, 0)
    plsc.subcore_barrier()
    pltpu.sync_copy(acc.at[pl.ds(s * ROWS_PER_TILE, ROWS_PER_TILE)],
                    out_hbm.at[pl.ds(base, ROWS_PER_TILE)])


@functools.cache
def _get_sc_step():
    return pl.kernel(
        _sc_step_body,
        out_type=jax.ShapeDtypeStruct((NPAD, H), jnp.float32),
        mesh=plsc.VectorSubcoreMesh(core_axis_name="c", subcore_axis_name="s",
                                    num_cores=NUM_CORES, num_subcores=NUM_TILES),
        scratch_types=[
            pltpu.VMEM_SHARED((ACC_ROWS, H), jnp.float32),
            pltpu.VMEM((NCHUNK, CHUNK), jnp.int32),
            pltpu.VMEM((NCHUNK, CHUNK), jnp.int32),
            pltpu.VMEM((CHUNK, H), jnp.float32),
            pltpu.SemaphoreType.DMA,
        ],
        compiler_params=pltpu.CompilerParams(use_tc_tiling_on_sc=False),
    )


_SELU_SCALE = 1.0507009873554805
_SELU_ALPHA = 1.6732632423543772


def _bn_body(h_ref, b1_ref, g_ref, bt_ref, o_ref):
    h = h_ref[...]
    mask = (lax.broadcasted_iota(jnp.int32, (NPAD, 1), 0) < N).astype(jnp.float32)
    hb = (h + b1_ref[...]) * mask
    mean = jnp.sum(hb, axis=0, keepdims=True) / N
    ctr = (hb - mean) * mask
    var = jnp.sum(ctr * ctr, axis=0, keepdims=True) / N
    z = (hb - mean) * lax.rsqrt(var + 1e-5) * g_ref[...] + bt_ref[...]
    act = _SELU_SCALE * jnp.where(z > 0, z, _SELU_ALPHA * (jnp.exp(z) - 1.0))
    o_ref[...] = act * mask


_bn_call = pl.pallas_call(
    _bn_body,
    out_shape=jax.ShapeDtypeStruct((NPAD, H), jnp.float32),
)


def _out_body(h_ref, w2_ref, b2_ref, o_ref):
    h = h_ref[...]
    logits = lax.dot_general(h, w2_ref[...], (((1,), (1,)), ((), ())),
                             preferred_element_type=jnp.float32) + b2_ref[...]
    m = jnp.max(logits, axis=1, keepdims=True)
    e = jnp.exp(logits - m)
    p = e / jnp.sum(e, axis=1, keepdims=True)
    o_ref[...] = p[:N, :]


_out_call = pl.pallas_call(
    _out_body,
    out_shape=jax.ShapeDtypeStruct((N, C), jnp.float32),
)


def _to_slabs(a, fill):
    a = a.reshape(NUM_TILES, EDGES_PER_TILE)
    a = jnp.pad(a, ((0, 0), (0, EPT_PAD - EDGES_PER_TILE)), constant_values=fill)
    return a.reshape(NUM_TILES, NCHUNK, CHUNK)


def kernel(x, edge_index, W1, b1, gamma, beta, W2, b2):
    x_pad = jnp.pad(x, ((0, NPAD - N), (0, 0)))
    src2d = edge_index[0].reshape(E // 128, 128)
    dst2d = edge_index[1].reshape(E // 128, 128)
    y, srcg, d0, d1 = _prep_call(x_pad, W1, src2d, dst2d)

    srcg_t = _to_slabs(srcg.reshape(-1), ZERO_ROW)
    dstl_t = jnp.stack([_to_slabs(d0.reshape(-1), DUMMY_DST),
                        _to_slabs(d1.reshape(-1), DUMMY_DST)])

    sc_step = _get_sc_step()
    h = sc_step(y, srcg_t, dstl_t)
    h = sc_step(h, srcg_t, dstl_t)
    h = _bn_call(h, b1.reshape(1, H), gamma.reshape(1, H), beta.reshape(1, H))
    h = sc_step(h, srcg_t, dstl_t)
    return _out_call(h, W2, b2.reshape(1, C))
